# baseline (speedup 1.0000x reference)
import functools

import jax
import jax.numpy as jnp
from jax import lax
from jax.experimental import pallas as pl
import jax.experimental.pallas.tpu as pltpu

B, CH, HW = 8, 3, 224
CF, NP, NBLK, K = 320, 196, 16, 9
NT = B * NP

_WTYPES = (
    ('il1_w1', (CF, CF)), ('il1_w2', (CF, CF)),
    ('ol1_w1', (CF, CF)), ('ol1_w2', (CF, CF)),
    ('il2_w1', (4 * CF, CF)), ('il2_w2', (CF, 4 * CF)),
    ('ol2_w1', (4 * CF, CF)), ('ol2_w2', (CF, 4 * CF)),
    ('fc_w', (CF, 2 * CF)),
    ('il1_b1', (CF,)), ('il1_b2', (CF,)),
    ('ol1_b1', (CF,)), ('ol1_b2', (CF,)),
    ('il2_b1', (4 * CF,)), ('il2_b2', (CF,)),
    ('ol2_b1', (4 * CF,)), ('ol2_b2', (CF,)),
    ('fc_b', (CF,)),
)
_NW = len(_WTYPES)

_SQRT_HALF = 0.7071067811865476


def _gelu(x):
    return 0.5 * x * (1.0 + lax.erf(x * _SQRT_HALF))


def _vig_blocks_kernel(*refs):
    x0_ref = refs[0]
    pose_ref = refs[1]
    wall = [refs[2 + blk * _NW: 2 + (blk + 1) * _NW] for blk in range(NBLK)]
    out_ref = refs[2 + NBLK * _NW]
    scr = refs[3 + NBLK * _NW: 3 + NBLK * _NW + _NW]
    sem = refs[3 + NBLK * _NW + _NW]
    xs_scratch = refs[4 + NBLK * _NW + _NW]

    i = pl.program_id(0)

    def issue(blk, slot):
        for t in range(_NW):
            pltpu.make_async_copy(wall[blk][t], scr[t].at[slot],
                                  sem.at[slot, t]).start()

    @pl.when(i == 0)
    def _init():
        xs_scratch[...] = (jnp.swapaxes(x0_ref[...], 1, 2)
                          + pose_ref[...][None, :, :])
        issue(0, 0)

    for blk in range(1, NBLK):
        @pl.when(i == blk - 1)
        def _prefetch(blk=blk):
            issue(blk, blk % 2)

    for slot in range(2):
        @pl.when(lax.rem(i, 2) == slot)
        def _drain(slot=slot):
            for t in range(_NW):
                pltpu.make_async_copy(wall[0][t], scr[t].at[slot],
                                      sem.at[slot, t]).wait()

    s_ = lax.rem(i, 2)
    (il1_w1, il1_w2, ol1_w1, ol1_w2, il2_w1, il2_w2, ol2_w1, ol2_w2,
     fc_w, il1_b1, il1_b2, ol1_b1, ol1_b2, il2_b1, il2_b2, ol2_b1,
     ol2_b2, fc_b) = [sc[s_] for sc in scr]

    def dot(a, b):
        return jnp.dot(a, b, preferred_element_type=jnp.float32)

    def dott(a, w):
        return lax.dot_general(a.astype(jnp.bfloat16), w.astype(jnp.bfloat16),
                               (((1,), (1,)), ((), ())),
                               preferred_element_type=jnp.float32)

    def tln(h, w1, b1, w2, b2):
        return dott(_gelu(dott(h, w1) + b1), w2) + b2

    x = xs_scratch[...]
    xf = x.reshape(NT, CF)
    x1f = tln(xf, il1_w1, il1_b1, il1_w2, il1_b2)
    x1 = x1f.reshape(B, NP, CF)

    t_rows = []
    for b in range(B):
        xb = x[b]
        s = lax.dot_general(xb, xb, (((1,), (1,)), ((), ())),
                            preferred_element_type=jnp.float32)
        tb = jnp.full((NP, CF), -jnp.inf, jnp.float32)
        for _ in range(K):
            m = jnp.max(s, axis=1, keepdims=True)
            oh = (s >= m).astype(jnp.float32)
            s = jnp.where(s >= m, -jnp.inf, s)
            tb = jnp.maximum(tb, dot(oh, x1[b]))
        t_rows.append(tb)
    t = jnp.stack(t_rows).reshape(NT, CF) - x1f

    r = lax.broadcasted_iota(jnp.int32, (2 * CF, CF), 0)
    c = lax.broadcasted_iota(jnp.int32, (2 * CF, CF), 1)
    wx = dot(fc_w, (r == 2 * c).astype(jnp.float32))
    wt = dot(fc_w, (r == 2 * c + 1).astype(jnp.float32))

    y = dott(x1f, wx) + dott(t, wt) + fc_b
    y = tln(_gelu(y), ol1_w1, ol1_b1, ol1_w2, ol1_b2)
    xn = y + xf
    z = tln(_gelu(tln(xn, il2_w1, il2_b1, il2_w2, il2_b2)),
            ol2_w1, ol2_b1, ol2_w2, ol2_b2)
    xout = (z + xn).reshape(B, NP, CF)
    xs_scratch[...] = xout

    @pl.when(i == NBLK - 1)
    def _fin():
        out_ref[...] = xout


@functools.partial(jax.jit, static_argnames=('interpret',))
def _vig_blocks(x0, pose, wlist, interpret=False):
    in_specs = [pl.BlockSpec(x0.shape, lambda i: (0, 0, 0)),
                pl.BlockSpec(pose.shape, lambda i: (0, 0))]
    in_specs += [pl.BlockSpec(memory_space=pl.ANY)] * (NBLK * _NW)
    out_shape = (B, NP, CF)
    return pl.pallas_call(
        _vig_blocks_kernel,
        grid=(NBLK,),
        in_specs=in_specs,
        out_specs=pl.BlockSpec(out_shape, lambda i: (0, 0, 0)),
        out_shape=jax.ShapeDtypeStruct(out_shape, jnp.float32),
        scratch_shapes=(
            [pltpu.VMEM((2,) + shp, jnp.float32) for _, shp in _WTYPES]
            + [pltpu.SemaphoreType.DMA((2, _NW)),
               pltpu.VMEM((B, NP, CF), jnp.float32)]
        ),
        interpret=interpret,
    )(x0, pose, *wlist)


def _stem(x, params, interpret=False):
    del interpret
    for i, sp in enumerate(params['stem']):
        y = lax.conv_general_dilated(
            x, sp['w'], (2, 2), [(1, 1), (1, 1)],
            dimension_numbers=('NCHW', 'OIHW', 'NCHW'))
        y = y + sp['b'][None, :, None, None]
        y = lax.reduce_window(y, -jnp.inf, lax.max, (1, 1, 3, 3),
                              (1, 1, 1, 1), [(0, 0), (0, 0), (1, 1), (1, 1)])
        m = jnp.mean(y, axis=(0, 2, 3), keepdims=True)
        v = jnp.var(y, axis=(0, 2, 3), keepdims=True)
        y = (y - m) / jnp.sqrt(v + 1e-5) * sp['g'][None, :, None, None] \
            + sp['be'][None, :, None, None]
        x = _gelu(y) if i < 3 else y
    return x


def kernel(x, params, interpret=False):
    x = _stem(x, params, interpret=interpret)
    Bb, C, H, W = x.shape
    x = x.reshape(Bb, C, H * W)
    wlist = [p[name] for p in params['blocks'] for name, _ in _WTYPES]
    return _vig_blocks(x, params['pose'], wlist, interpret=interpret)


# speedup vs baseline: 1.0027x; 1.0027x over previous
import functools

import jax
import jax.numpy as jnp
from jax import lax
from jax.experimental import pallas as pl
import jax.experimental.pallas.tpu as pltpu

B, CH, HW = 8, 3, 224
CF, NP, NBLK, K = 320, 196, 16, 9
NT = B * NP

_WTYPES = (
    ('il1_w1', (CF, CF)), ('il1_w2', (CF, CF)),
    ('ol1_w1', (CF, CF)), ('ol1_w2', (CF, CF)),
    ('il2_w1', (4 * CF, CF)), ('il2_w2', (CF, 4 * CF)),
    ('ol2_w1', (4 * CF, CF)), ('ol2_w2', (CF, 4 * CF)),
    ('fc_w', (CF, 2 * CF)),
    ('il1_b1', (CF,)), ('il1_b2', (CF,)),
    ('ol1_b1', (CF,)), ('ol1_b2', (CF,)),
    ('il2_b1', (4 * CF,)), ('il2_b2', (CF,)),
    ('ol2_b1', (4 * CF,)), ('ol2_b2', (CF,)),
    ('fc_b', (CF,)),
)
_NW = len(_WTYPES)

_SQRT_HALF = 0.7071067811865476


def _gelu(x):
    return 0.5 * x * (1.0 + lax.erf(x * _SQRT_HALF))


def _vig_blocks_kernel(*refs):
    x0_ref = refs[0]
    pose_ref = refs[1]
    wall = [refs[2 + blk * _NW: 2 + (blk + 1) * _NW] for blk in range(NBLK)]
    out_ref = refs[2 + NBLK * _NW]
    scr = refs[3 + NBLK * _NW: 3 + NBLK * _NW + _NW]
    sem = refs[3 + NBLK * _NW + _NW]
    xs_scratch = refs[4 + NBLK * _NW + _NW]

    i = pl.program_id(0)

    def issue(blk, slot):
        for t in range(_NW):
            pltpu.make_async_copy(wall[blk][t], scr[t].at[slot],
                                  sem.at[slot, t]).start()

    @pl.when(i == 0)
    def _init():
        xs_scratch[...] = (jnp.swapaxes(x0_ref[...], 1, 2)
                          + pose_ref[...][None, :, :])
        issue(0, 0)

    for blk in range(1, NBLK):
        @pl.when(i == blk - 1)
        def _prefetch(blk=blk):
            issue(blk, blk % 2)

    for slot in range(2):
        @pl.when(lax.rem(i, 2) == slot)
        def _drain(slot=slot):
            for t in range(_NW):
                pltpu.make_async_copy(wall[0][t], scr[t].at[slot],
                                      sem.at[slot, t]).wait()

    s_ = lax.rem(i, 2)
    (il1_w1, il1_w2, ol1_w1, ol1_w2, il2_w1, il2_w2, ol2_w1, ol2_w2,
     fc_w, il1_b1, il1_b2, ol1_b1, ol1_b2, il2_b1, il2_b2, ol2_b1,
     ol2_b2, fc_b) = [sc[s_] for sc in scr]

    def dot(a, b):
        return jnp.dot(a, b, preferred_element_type=jnp.float32)

    def dott(a, w):
        return lax.dot_general(a, w, (((1,), (1,)), ((), ())),
                               preferred_element_type=jnp.float32)

    def tln(h, w1, b1, w2, b2):
        return dott(_gelu(dott(h, w1) + b1), w2) + b2

    x = xs_scratch[...]
    xf = x.reshape(NT, CF)
    x1f = tln(xf, il1_w1, il1_b1, il1_w2, il1_b2)
    x1 = x1f.reshape(B, NP, CF)

    t_rows = []
    for b in range(B):
        xb = x[b]
        s = dott(xb, xb)
        ohs = []
        for k in range(K):
            m = jnp.max(s, axis=1, keepdims=True)
            sel = s >= m
            ohs.append(sel.astype(jnp.float32))
            if k < K - 1:
                s = jnp.where(sel, -jnp.inf, s)
        nb = dot(jnp.concatenate(ohs, axis=0), x1[b])
        tb = nb.reshape(K, NP, CF).max(axis=0)
        t_rows.append(tb)
    t = jnp.stack(t_rows).reshape(NT, CF) - x1f

    r = lax.broadcasted_iota(jnp.int32, (2 * CF, CF), 0)
    c = lax.broadcasted_iota(jnp.int32, (2 * CF, CF), 1)
    wx = dot(fc_w, (r == 2 * c).astype(jnp.float32))
    wt = dot(fc_w, (r == 2 * c + 1).astype(jnp.float32))

    y = dott(x1f, wx) + dott(t, wt) + fc_b
    y = tln(_gelu(y), ol1_w1, ol1_b1, ol1_w2, ol1_b2)
    xn = y + xf
    z = tln(_gelu(tln(xn, il2_w1, il2_b1, il2_w2, il2_b2)),
            ol2_w1, ol2_b1, ol2_w2, ol2_b2)
    xout = (z + xn).reshape(B, NP, CF)
    xs_scratch[...] = xout

    @pl.when(i == NBLK - 1)
    def _fin():
        out_ref[...] = xout


@functools.partial(jax.jit, static_argnames=('interpret',))
def _vig_blocks(x0, pose, wlist, interpret=False):
    in_specs = [pl.BlockSpec(x0.shape, lambda i: (0, 0, 0)),
                pl.BlockSpec(pose.shape, lambda i: (0, 0))]
    in_specs += [pl.BlockSpec(memory_space=pl.ANY)] * (NBLK * _NW)
    out_shape = (B, NP, CF)
    return pl.pallas_call(
        _vig_blocks_kernel,
        grid=(NBLK,),
        in_specs=in_specs,
        out_specs=pl.BlockSpec(out_shape, lambda i: (0, 0, 0)),
        out_shape=jax.ShapeDtypeStruct(out_shape, jnp.float32),
        scratch_shapes=(
            [pltpu.VMEM((2,) + shp, jnp.float32) for _, shp in _WTYPES]
            + [pltpu.SemaphoreType.DMA((2, _NW)),
               pltpu.VMEM((B, NP, CF), jnp.float32)]
        ),
        interpret=interpret,
    )(x0, pose, *wlist)


def _stem(x, params, interpret=False):
    del interpret
    for i, sp in enumerate(params['stem']):
        y = lax.conv_general_dilated(
            x, sp['w'], (2, 2), [(1, 1), (1, 1)],
            dimension_numbers=('NCHW', 'OIHW', 'NCHW'))
        y = y + sp['b'][None, :, None, None]
        y = lax.reduce_window(y, -jnp.inf, lax.max, (1, 1, 3, 3),
                              (1, 1, 1, 1), [(0, 0), (0, 0), (1, 1), (1, 1)])
        m = jnp.mean(y, axis=(0, 2, 3), keepdims=True)
        v = jnp.var(y, axis=(0, 2, 3), keepdims=True)
        y = (y - m) / jnp.sqrt(v + 1e-5) * sp['g'][None, :, None, None] \
            + sp['be'][None, :, None, None]
        x = _gelu(y) if i < 3 else y
    return x


def kernel(x, params, interpret=False):
    x = _stem(x, params, interpret=interpret)
    Bb, C, H, W = x.shape
    x = x.reshape(Bb, C, H * W)
    wlist = [p[name] for p in params['blocks'] for name, _ in _WTYPES]
    return _vig_blocks(x, params['pose'], wlist, interpret=interpret)


# speedup vs baseline: 1.0080x; 1.0053x over previous
import jax
import jax.numpy as jnp
from jax import lax
from jax.experimental import pallas as pl
import jax.experimental.pallas.tpu as pltpu

B, CH, HW = 8, 3, 224
CF, NP, NBLK, K = 320, 196, 16, 9
NT = B * NP

_WTYPES = (
    ('il1_w1', (CF, CF)), ('il1_w2', (CF, CF)),
    ('ol1_w1', (CF, CF)), ('ol1_w2', (CF, CF)),
    ('il2_w1', (4 * CF, CF)), ('il2_w2', (CF, 4 * CF)),
    ('ol2_w1', (4 * CF, CF)), ('ol2_w2', (CF, 4 * CF)),
    ('fc_w', (CF, 2 * CF)),
    ('il1_b1', (CF,)), ('il1_b2', (CF,)),
    ('ol1_b1', (CF,)), ('ol1_b2', (CF,)),
    ('il2_b1', (4 * CF,)), ('il2_b2', (CF,)),
    ('ol2_b1', (4 * CF,)), ('ol2_b2', (CF,)),
    ('fc_b', (CF,)),
)
_NW = len(_WTYPES)

_SQRT_HALF = 0.7071067811865476


def _gelu(x):
    return 0.5 * x * (1.0 + lax.erf(x * _SQRT_HALF))


def _vig_blocks_kernel(*refs):
    x0_ref = refs[0]
    pose_ref = refs[1]
    wall = [refs[2 + blk * _NW: 2 + (blk + 1) * _NW] for blk in range(NBLK)]
    out_ref = refs[2 + NBLK * _NW]
    scr = refs[3 + NBLK * _NW: 3 + NBLK * _NW + _NW]
    sem = refs[3 + NBLK * _NW + _NW]
    xs_scratch = refs[4 + NBLK * _NW + _NW]

    i = pl.program_id(0)

    def issue(blk, slot):
        for t in range(_NW):
            pltpu.make_async_copy(wall[blk][t], scr[t].at[slot],
                                  sem.at[slot, t]).start()

    @pl.when(i == 0)
    def _init():
        xs_scratch[...] = (jnp.swapaxes(x0_ref[...], 1, 2)
                          + pose_ref[...][None, :, :])
        issue(0, 0)

    for blk in range(1, NBLK):
        @pl.when(i == blk - 1)
        def _prefetch(blk=blk):
            issue(blk, blk % 2)

    for slot in range(2):
        @pl.when(lax.rem(i, 2) == slot)
        def _drain(slot=slot):
            for t in range(_NW):
                pltpu.make_async_copy(wall[0][t], scr[t].at[slot],
                                      sem.at[slot, t]).wait()

    s_ = lax.rem(i, 2)
    (il1_w1, il1_w2, ol1_w1, ol1_w2, il2_w1, il2_w2, ol2_w1, ol2_w2,
     fc_w, il1_b1, il1_b2, ol1_b1, ol1_b2, il2_b1, il2_b2, ol2_b1,
     ol2_b2, fc_b) = [sc[s_] for sc in scr]

    def dot(a, b):
        return jnp.dot(a, b, preferred_element_type=jnp.float32)

    def dott(a, w):
        return lax.dot_general(a, w, (((1,), (1,)), ((), ())),
                               preferred_element_type=jnp.float32)

    def tln(h, w1, b1, w2, b2):
        return dott(_gelu(dott(h, w1) + b1), w2) + b2

    x = xs_scratch[...]
    xf = x.reshape(NT, CF)
    x1f = tln(xf, il1_w1, il1_b1, il1_w2, il1_b2)
    x1 = x1f.reshape(B, NP, CF)

    t_rows = []
    for b in range(B):
        xb = x[b]
        s = dott(xb, xb)
        tb = jnp.full((NP, CF), -jnp.inf, jnp.float32)
        for _ in range(K):
            m = jnp.max(s, axis=1, keepdims=True)
            oh = (s >= m).astype(jnp.float32)
            s = jnp.where(s >= m, -jnp.inf, s)
            tb = jnp.maximum(tb, dot(oh, x1[b]))
        t_rows.append(tb)
    t = jnp.stack(t_rows).reshape(NT, CF) - x1f

    r = lax.broadcasted_iota(jnp.int32, (2 * CF, CF), 0)
    c = lax.broadcasted_iota(jnp.int32, (2 * CF, CF), 1)
    wx = dot(fc_w, (r == 2 * c).astype(jnp.float32))
    wt = dot(fc_w, (r == 2 * c + 1).astype(jnp.float32))

    y = dott(x1f, wx) + dott(t, wt) + fc_b
    y = tln(_gelu(y), ol1_w1, ol1_b1, ol1_w2, ol1_b2)
    xn = y + xf
    z = tln(_gelu(tln(xn, il2_w1, il2_b1, il2_w2, il2_b2)),
            ol2_w1, ol2_b1, ol2_w2, ol2_b2)
    xout = (z + xn).reshape(B, NP, CF)
    xs_scratch[...] = xout

    @pl.when(i == NBLK - 1)
    def _fin():
        out_ref[...] = xout


def _vig_blocks(x0, pose, wlist):
    in_specs = [pl.BlockSpec(x0.shape, lambda i: (0, 0, 0)),
                pl.BlockSpec(pose.shape, lambda i: (0, 0))]
    in_specs += [pl.BlockSpec(memory_space=pl.ANY)] * (NBLK * _NW)
    out_shape = (B, NP, CF)
    return pl.pallas_call(
        _vig_blocks_kernel,
        grid=(NBLK,),
        in_specs=in_specs,
        out_specs=pl.BlockSpec(out_shape, lambda i: (0, 0, 0)),
        out_shape=jax.ShapeDtypeStruct(out_shape, jnp.float32),
        scratch_shapes=(
            [pltpu.VMEM((2,) + shp, jnp.float32) for _, shp in _WTYPES]
            + [pltpu.SemaphoreType.DMA((2, _NW)),
               pltpu.VMEM((B, NP, CF), jnp.float32)]
        ),
    )(x0, pose, *wlist)


def _stem(x, params):
    for i, sp in enumerate(params['stem']):
        y = lax.conv_general_dilated(
            x, sp['w'], (2, 2), [(1, 1), (1, 1)],
            dimension_numbers=('NCHW', 'OIHW', 'NCHW'))
        y = y + sp['b'][None, :, None, None]
        y = lax.reduce_window(y, -jnp.inf, lax.max, (1, 1, 3, 3),
                              (1, 1, 1, 1), [(0, 0), (0, 0), (1, 1), (1, 1)])
        m = jnp.mean(y, axis=(0, 2, 3), keepdims=True)
        v = jnp.var(y, axis=(0, 2, 3), keepdims=True)
        y = (y - m) / jnp.sqrt(v + 1e-5) * sp['g'][None, :, None, None] \
            + sp['be'][None, :, None, None]
        x = _gelu(y) if i < 3 else y
    return x


def kernel(x, params):
    x = _stem(x, params)
    Bb, C, H, W = x.shape
    x = x.reshape(Bb, C, H * W)
    wlist = [p[name] for p in params['blocks'] for name, _ in _WTYPES]
    return _vig_blocks(x, params['pose'], wlist)
